# Initial kernel scaffold; baseline (speedup 1.0000x reference)
import jax
import jax.numpy as jnp
from jax.experimental import pallas as pl


def kernel(x, pos, edge_attr, node_sigma_emb, edge_index, W_ne1, b_ne1, W_ne2, b_ne2, W_ee1, b_ee1, W_ee2, b_ee2, W_g1, b_g1, W_g2, b_g2, W_tp):
    raise NotImplementedError("write your pallas kernel here")


# speedup vs baseline: 2.0225x; 2.0225x over previous
import functools

import jax
import jax.numpy as jnp
from jax import lax
from jax.experimental import pallas as pl
from jax.experimental.pallas import tpu as pltpu
from jax.experimental.pallas import tpu_sc as plsc

N_NODES = 10000
N_EDGES = 160000
NS = 48

SC_CORES = 2
SC_SUBCORES = 16
SC_WORKERS = SC_CORES * SC_SUBCORES
EDGES_PER_WORKER = N_EDGES // SC_WORKERS
GCHUNK = 1000
SCHUNK = 40
SCHUNKS_PER_WORKER = EDGES_PER_WORKER // SCHUNK
NODES_PER_TILE = N_NODES // SC_SUBCORES

EB = 1000
NB = 2000

_SQ3 = 3.0 ** 0.5
_SQ15 = 15.0 ** 0.5
_SQ5 = 5.0 ** 0.5
_SMEAR_STEP = 5.0 / 31.0
_SMEAR_COEFF = -0.5 / (_SMEAR_STEP * _SMEAR_STEP)


def _node_mlp_body(x_ref, sg_ref, w1_ref, b1_ref, w2_ref, b2_ref, h_ref):
    cat = jnp.concatenate([x_ref[...], sg_ref[...]], axis=1)
    a = jnp.dot(cat, w1_ref[...], preferred_element_type=jnp.float32) + b1_ref[...]
    a = jnp.maximum(a, 0.0)
    h_ref[...] = jnp.dot(a, w2_ref[...], preferred_element_type=jnp.float32) + b2_ref[...]


def _node_mlp(x, sig, w1, b1, w2, b2):
    grid = N_NODES // NB
    full = lambda shape: pl.BlockSpec(shape, lambda i: (0, 0))
    return pl.pallas_call(
        _node_mlp_body,
        grid=(grid,),
        in_specs=[
            pl.BlockSpec((NB, 16), lambda i: (i, 0)),
            pl.BlockSpec((NB, 32), lambda i: (i, 0)),
            full((48, NS)),
            full((1, NS)),
            full((NS, NS)),
            full((1, NS)),
        ],
        out_specs=pl.BlockSpec((NB, NS), lambda i: (i, 0)),
        out_shape=jax.ShapeDtypeStruct((N_NODES, NS), jnp.float32),
    )(x, sig, w1, b1, w2, b2)


def _gather_body(h_hbm, sig_hbm, pos_hbm, src_hbm, dst_hbm,
                 hs_hbm, hd_hbm, sgs_hbm, pss_hbm, pds_hbm,
                 ichunk, b48, b32, b16, sem):
    wid = lax.axis_index("c") * SC_SUBCORES + lax.axis_index("s")
    base = wid * EDGES_PER_WORKER
    for k in range(EDGES_PER_WORKER // GCHUNK):
        sl = pl.ds(base + k * GCHUNK, GCHUNK)
        pltpu.sync_copy(src_hbm.at[sl], ichunk)
        pltpu.async_copy(h_hbm.at[ichunk], b48, sem).wait()
        pltpu.sync_copy(b48, hs_hbm.at[sl])
        pltpu.async_copy(sig_hbm.at[ichunk], b32, sem).wait()
        pltpu.sync_copy(b32, sgs_hbm.at[sl])
        pltpu.async_copy(pos_hbm.at[ichunk], b16, sem).wait()
        pltpu.sync_copy(b16, pss_hbm.at[sl])
        pltpu.sync_copy(dst_hbm.at[sl], ichunk)
        pltpu.async_copy(h_hbm.at[ichunk], b48, sem).wait()
        pltpu.sync_copy(b48, hd_hbm.at[sl])
        pltpu.async_copy(pos_hbm.at[ichunk], b16, sem).wait()
        pltpu.sync_copy(b16, pds_hbm.at[sl])


def _sc_gather(h, sig, pos16, src, dst):
    mesh = plsc.VectorSubcoreMesh(core_axis_name="c", subcore_axis_name="s")
    f32 = jnp.float32
    out_type = [
        jax.ShapeDtypeStruct((N_EDGES, 48), f32),
        jax.ShapeDtypeStruct((N_EDGES, 48), f32),
        jax.ShapeDtypeStruct((N_EDGES, 32), f32),
        jax.ShapeDtypeStruct((N_EDGES, 16), f32),
        jax.ShapeDtypeStruct((N_EDGES, 16), f32),
    ]
    scratch = [
        pltpu.VMEM((GCHUNK,), jnp.int32),
        pltpu.VMEM((GCHUNK, 48), f32),
        pltpu.VMEM((GCHUNK, 32), f32),
        pltpu.VMEM((GCHUNK, 16), f32),
        pltpu.SemaphoreType.DMA,
    ]
    fn = pl.kernel(_gather_body, out_type=out_type, mesh=mesh,
                   scratch_types=scratch,
                   compiler_params=pltpu.CompilerParams(
                       use_tc_tiling_on_sc=False))
    return fn(h, sig, pos16, src, dst)


def _edge_body(ea_ref, sgs_ref, ps_ref, pd_ref, hs_ref, hd_ref,
               wee1_ref, bee1_ref, wee2_ref, bee2_ref,
               wg1_ref, bg1_ref, wg2_ref, bg2_ref, wcat_ref, out_ref):
    f32 = jnp.float32
    ps = ps_ref[...]
    pd = pd_ref[...]
    vx = pd[:, 0:1] - ps[:, 0:1]
    vy = pd[:, 1:2] - ps[:, 1:2]
    vz = pd[:, 2:3] - ps[:, 2:3]
    d = jnp.sqrt(vx * vx + vy * vy + vz * vz + 1e-12)
    ux = vx / d
    uy = vy / d
    uz = vz / d

    offs = lax.broadcasted_iota(jnp.int32, (1, 32), 1).astype(f32) * _SMEAR_STEP
    dd = d - offs
    demb = jnp.exp(_SMEAR_COEFF * dd * dd)

    hs = hs_ref[...]
    hd = hd_ref[...]

    e_in = jnp.concatenate([ea_ref[...], sgs_ref[...], demb], axis=1)
    e1 = jnp.dot(e_in, wee1_ref[...], preferred_element_type=f32) + bee1_ref[...]
    e1 = jnp.maximum(e1, 0.0)
    e2 = jnp.dot(e1, wee2_ref[...], preferred_element_type=f32) + bee2_ref[...]

    g_in = jnp.concatenate([e2, hs, hd], axis=1)
    g1 = jnp.dot(g_in, wg1_ref[...], preferred_element_type=f32) + bg1_ref[...]
    g1 = jnp.maximum(g1, 0.0)
    w9 = jnp.dot(g1, wg2_ref[...], preferred_element_type=f32) + bg2_ref[...]

    sh = [
        jnp.ones_like(d),
        _SQ3 * ux,
        _SQ3 * uy,
        _SQ3 * uz,
        _SQ15 * ux * uy,
        _SQ15 * uy * uz,
        (_SQ5 / 2.0) * (3.0 * uz * uz - 1.0),
        _SQ15 * ux * uz,
        (_SQ15 / 2.0) * (ux * ux - uy * uy),
    ]

    p = jnp.dot(hs, wcat_ref[...], preferred_element_type=f32)
    acc = (sh[0] * w9[:, 0:1]) * p[:, 0:48]
    for j in range(1, 9):
        acc = acc + (sh[j] * w9[:, j:j + 1]) * p[:, j * 48:(j + 1) * 48]
    out_ref[...] = acc


def _edge_compute(ea, sgs, pss, pds, hs, hd, wee1, bee1, wee2, bee2,
                  wg1, bg1, wg2, bg2, wcat):
    grid = N_EDGES // EB
    full = lambda shape: pl.BlockSpec(shape, lambda i: (0, 0))
    eb = lambda d: pl.BlockSpec((EB, d), lambda i: (i, 0))
    return pl.pallas_call(
        _edge_body,
        grid=(grid,),
        in_specs=[
            eb(4), eb(32), eb(16), eb(16), eb(48), eb(48),
            full((68, NS)), full((1, NS)), full((NS, NS)), full((1, NS)),
            full((3 * NS, 3 * NS)), full((1, 3 * NS)),
            full((3 * NS, 9)), full((1, 9)),
            full((NS, NS * 9)),
        ],
        out_specs=pl.BlockSpec((EB, NS), lambda i: (i, 0)),
        out_shape=jax.ShapeDtypeStruct((N_EDGES, NS), jnp.float32),
    )(ea, sgs, pss, pds, hs, hd, wee1, bee1, wee2, bee2, wg1, bg1, wg2,
      bg2, wcat)


def _scatter_body(oute_hbm, dst2d_hbm, zero_hbm, parts_hbm,
                  shared, idx2d, db):
    cid = lax.axis_index("c")
    sid = lax.axis_index("s")
    wid = cid * SC_SUBCORES + sid
    rows = pl.ds(sid * NODES_PER_TILE, NODES_PER_TILE)

    pltpu.sync_copy(zero_hbm.at[rows], shared.at[rows])
    pltpu.sync_copy(
        dst2d_hbm.at[pl.ds(wid * SCHUNKS_PER_WORKER, SCHUNKS_PER_WORKER)],
        idx2d)
    plsc.subcore_barrier()

    base = wid * EDGES_PER_WORKER
    for k in range(SCHUNKS_PER_WORKER):
        pltpu.sync_copy(oute_hbm.at[pl.ds(base + k * SCHUNK, SCHUNK)], db)
        pltpu.sync_copy(db, shared.at[idx2d.at[k]], add=True)
    plsc.subcore_barrier()

    pltpu.sync_copy(shared.at[rows], parts_hbm.at[cid, rows])


def _sc_scatter(oute, dst2d, zeros_hbm):
    mesh = plsc.VectorSubcoreMesh(core_axis_name="c", subcore_axis_name="s")
    f32 = jnp.float32
    out_type = jax.ShapeDtypeStruct((SC_CORES, N_NODES, NS), f32)
    scratch = [
        pltpu.VMEM_SHARED((N_NODES, NS), f32),
        pltpu.VMEM((SCHUNKS_PER_WORKER, SCHUNK), jnp.int32),
        pltpu.VMEM((SCHUNK, NS), f32),
    ]
    fn = pl.kernel(_scatter_body, out_type=out_type, mesh=mesh,
                   scratch_types=scratch,
                   compiler_params=pltpu.CompilerParams(
                       use_tc_tiling_on_sc=False))
    return fn(oute, dst2d, zeros_hbm)


def _combine_body(h_ref, p0_ref, p1_ref, out_ref):
    out_ref[...] = h_ref[...] + (p0_ref[...] + p1_ref[...]) * 0.25


def _combine(h, p0, p1):
    grid = N_NODES // NB
    spec = pl.BlockSpec((NB, NS), lambda i: (i, 0))
    return pl.pallas_call(
        _combine_body,
        grid=(grid,),
        in_specs=[spec, spec, spec],
        out_specs=spec,
        out_shape=jax.ShapeDtypeStruct((N_NODES, NS), jnp.float32),
    )(h, p0, p1)


def kernel(x, pos, edge_attr, node_sigma_emb, edge_index,
           W_ne1, b_ne1, W_ne2, b_ne2, W_ee1, b_ee1, W_ee2, b_ee2,
           W_g1, b_g1, W_g2, b_g2, W_tp):
    f32 = jnp.float32
    src = edge_index[0]
    dst = edge_index[1]

    pos16 = jnp.concatenate(
        [pos.astype(f32), jnp.zeros((N_NODES, 13), f32)], axis=1)
    wcat = W_tp.reshape(NS, NS * 9)
    b_ne1r = b_ne1.reshape(1, NS)
    b_ne2r = b_ne2.reshape(1, NS)
    b_ee1r = b_ee1.reshape(1, NS)
    b_ee2r = b_ee2.reshape(1, NS)
    b_g1r = b_g1.reshape(1, 3 * NS)
    b_g2r = b_g2.reshape(1, 9)

    h = _node_mlp(x, node_sigma_emb, W_ne1, b_ne1r, W_ne2, b_ne2r)

    hs, hd, sgs, pss, pds = _sc_gather(h, node_sigma_emb, pos16, src, dst)

    oute = _edge_compute(edge_attr, sgs, pss, pds, hs, hd,
                         W_ee1, b_ee1r, W_ee2, b_ee2r,
                         W_g1, b_g1r, W_g2, b_g2r, wcat)

    dst2d = dst.reshape(N_EDGES // SCHUNK, SCHUNK)
    zeros_hbm = jnp.zeros((N_NODES, NS), f32)
    parts = _sc_scatter(oute, dst2d, zeros_hbm)

    return _combine(h, parts[0], parts[1])


# speedup vs baseline: 2.7845x; 1.3768x over previous
import functools

import jax
import jax.numpy as jnp
from jax import lax
from jax.experimental import pallas as pl
from jax.experimental.pallas import tpu as pltpu
from jax.experimental.pallas import tpu_sc as plsc

N_NODES = 10000
N_EDGES = 160000
NS = 48

SC_CORES = 2
SC_SUBCORES = 16
SC_WORKERS = SC_CORES * SC_SUBCORES
EDGES_PER_WORKER = N_EDGES // SC_WORKERS
GCHUNK = 1000
SCHUNK = 40
SCHUNKS_PER_WORKER = EDGES_PER_WORKER // SCHUNK
NODES_PER_TILE = N_NODES // SC_SUBCORES

EB = 1000
NB = 2000

_SQ3 = 3.0 ** 0.5
_SQ15 = 15.0 ** 0.5
_SQ5 = 5.0 ** 0.5
_SMEAR_STEP = 5.0 / 31.0
_SMEAR_COEFF = -0.5 / (_SMEAR_STEP * _SMEAR_STEP)


def _node_mlp_body(x_ref, sg_ref, w1_ref, b1_ref, w2_ref, b2_ref, h_ref):
    cat = jnp.concatenate([x_ref[...], sg_ref[...]], axis=1)
    a = jnp.dot(cat, w1_ref[...], preferred_element_type=jnp.float32) + b1_ref[...]
    a = jnp.maximum(a, 0.0)
    h_ref[...] = jnp.dot(a, w2_ref[...], preferred_element_type=jnp.float32) + b2_ref[...]


def _node_mlp(x, sig, w1, b1, w2, b2):
    grid = N_NODES // NB
    full = lambda shape: pl.BlockSpec(shape, lambda i: (0, 0))
    return pl.pallas_call(
        _node_mlp_body,
        grid=(grid,),
        in_specs=[
            pl.BlockSpec((NB, 16), lambda i: (i, 0)),
            pl.BlockSpec((NB, 32), lambda i: (i, 0)),
            full((48, NS)),
            full((1, NS)),
            full((NS, NS)),
            full((1, NS)),
        ],
        out_specs=pl.BlockSpec((NB, NS), lambda i: (i, 0)),
        out_shape=jax.ShapeDtypeStruct((N_NODES, NS), jnp.float32),
    )(x, sig, w1, b1, w2, b2)


def _gather_body(h_hbm, sig_hbm, pos_hbm, src_hbm, dst_hbm,
                 hs_hbm, hd_hbm, sgs_hbm, pss_hbm, pds_hbm,
                 ichunk, b48, b32, b16, sem):
    wid = lax.axis_index("c") * SC_SUBCORES + lax.axis_index("s")
    base = wid * EDGES_PER_WORKER
    for k in range(EDGES_PER_WORKER // GCHUNK):
        sl = pl.ds(base + k * GCHUNK, GCHUNK)
        pltpu.sync_copy(src_hbm.at[sl], ichunk)
        pltpu.async_copy(h_hbm.at[ichunk], b48, sem).wait()
        pltpu.sync_copy(b48, hs_hbm.at[sl])
        pltpu.async_copy(sig_hbm.at[ichunk], b32, sem).wait()
        pltpu.sync_copy(b32, sgs_hbm.at[sl])
        pltpu.async_copy(pos_hbm.at[ichunk], b16, sem).wait()
        pltpu.sync_copy(b16, pss_hbm.at[sl])
        pltpu.sync_copy(dst_hbm.at[sl], ichunk)
        pltpu.async_copy(h_hbm.at[ichunk], b48, sem).wait()
        pltpu.sync_copy(b48, hd_hbm.at[sl])
        pltpu.async_copy(pos_hbm.at[ichunk], b16, sem).wait()
        pltpu.sync_copy(b16, pds_hbm.at[sl])


def _sc_gather(h, sig, pos16, src, dst):
    mesh = plsc.VectorSubcoreMesh(core_axis_name="c", subcore_axis_name="s")
    f32 = jnp.float32
    out_type = [
        jax.ShapeDtypeStruct((N_EDGES, 48), f32),
        jax.ShapeDtypeStruct((N_EDGES, 48), f32),
        jax.ShapeDtypeStruct((N_EDGES, 32), f32),
        jax.ShapeDtypeStruct((N_EDGES, 16), f32),
        jax.ShapeDtypeStruct((N_EDGES, 16), f32),
    ]
    scratch = [
        pltpu.VMEM((GCHUNK,), jnp.int32),
        pltpu.VMEM((GCHUNK, 48), f32),
        pltpu.VMEM((GCHUNK, 32), f32),
        pltpu.VMEM((GCHUNK, 16), f32),
        pltpu.SemaphoreType.DMA,
    ]
    fn = pl.kernel(_gather_body, out_type=out_type, mesh=mesh,
                   scratch_types=scratch,
                   compiler_params=pltpu.CompilerParams(
                       use_tc_tiling_on_sc=False))
    return fn(h, sig, pos16, src, dst)


def _edge_body(ea_ref, sgs_ref, ps_ref, pd_ref, hs_ref, hd_ref,
               wee1_ref, bee1_ref, wee2_ref, bee2_ref,
               wg1_ref, bg1_ref, wg2_ref, bg2_ref, wtp_ref,
               m1_ref, m2_ref, alin_ref, kc_ref, e48_ref, t9_ref, out_ref):
    f32 = jnp.float32
    v = pd_ref[...] - ps_ref[...]
    vv = v * v
    ones16 = jnp.full((16, 16), 1.0, f32)
    ones32 = jnp.full((16, 32), 1.0, f32)
    d2_16 = jnp.dot(vv, ones16, preferred_element_type=f32) + 1e-12
    d2_32 = jnp.dot(vv, ones32, preferred_element_type=f32) + 1e-12
    u = v * lax.rsqrt(d2_16)

    offs = lax.broadcasted_iota(jnp.int32, (1, 32), 1).astype(f32) * _SMEAR_STEP
    dd = jnp.sqrt(d2_32) - offs
    demb = jnp.exp(_SMEAR_COEFF * dd * dd)

    hs = hs_ref[...]
    hd = hd_ref[...]

    e_in = jnp.concatenate([ea_ref[...], sgs_ref[...], demb], axis=1)
    e1 = jnp.dot(e_in, wee1_ref[...], preferred_element_type=f32) + bee1_ref[...]
    e1 = jnp.maximum(e1, 0.0)
    e2 = jnp.dot(e1, wee2_ref[...], preferred_element_type=f32) + bee2_ref[...]

    g_in = jnp.concatenate([e2, hs, hd], axis=1)
    g1 = jnp.dot(g_in, wg1_ref[...], preferred_element_type=f32) + bg1_ref[...]
    g1 = jnp.maximum(g1, 0.0)
    w9 = jnp.dot(g1, wg2_ref[...], preferred_element_type=f32) + bg2_ref[...]

    sh9 = (jnp.dot(u, m1_ref[...], preferred_element_type=f32)
           * jnp.dot(u, m2_ref[...], preferred_element_type=f32)
           + jnp.dot(u, alin_ref[...], preferred_element_type=f32)
           + kc_ref[...])
    shw = sh9 * w9

    hse = jnp.dot(hs, e48_ref[...], preferred_element_type=f32)
    shwt = jnp.dot(shw, t9_ref[...], preferred_element_type=f32)
    out_ref[...] = jnp.dot(hse * shwt, wtp_ref[...],
                           preferred_element_type=f32)


def _edge_compute(ea, sgs, pss, pds, hs, hd, wee1, bee1, wee2, bee2,
                  wg1, bg1, wg2, bg2, wtp, m1, m2, alin, kc, e48, t9):
    grid = N_EDGES // EB
    full = lambda shape: pl.BlockSpec(shape, lambda i: (0, 0))
    eb = lambda d: pl.BlockSpec((EB, d), lambda i: (i, 0))
    return pl.pallas_call(
        _edge_body,
        grid=(grid,),
        in_specs=[
            eb(4), eb(32), eb(16), eb(16), eb(48), eb(48),
            full((68, NS)), full((1, NS)), full((NS, NS)), full((1, NS)),
            full((3 * NS, 3 * NS)), full((1, 3 * NS)),
            full((3 * NS, 9)), full((1, 9)),
            full((NS * 9, NS)),
            full((16, 9)), full((16, 9)), full((16, 9)), full((1, 9)),
            full((NS, NS * 9)), full((9, NS * 9)),
        ],
        out_specs=pl.BlockSpec((EB, NS), lambda i: (i, 0)),
        out_shape=jax.ShapeDtypeStruct((N_EDGES, NS), jnp.float32),
    )(ea, sgs, pss, pds, hs, hd, wee1, bee1, wee2, bee2, wg1, bg1, wg2,
      bg2, wtp, m1, m2, alin, kc, e48, t9)


def _scatter_body(oute_hbm, dst2d_hbm, zero_hbm, parts_hbm,
                  shared, idx2d, db):
    cid = lax.axis_index("c")
    sid = lax.axis_index("s")
    wid = cid * SC_SUBCORES + sid
    rows = pl.ds(sid * NODES_PER_TILE, NODES_PER_TILE)

    pltpu.sync_copy(zero_hbm.at[rows], shared.at[rows])
    pltpu.sync_copy(
        dst2d_hbm.at[pl.ds(wid * SCHUNKS_PER_WORKER, SCHUNKS_PER_WORKER)],
        idx2d)
    plsc.subcore_barrier()

    base = wid * EDGES_PER_WORKER
    for k in range(SCHUNKS_PER_WORKER):
        pltpu.sync_copy(oute_hbm.at[pl.ds(base + k * SCHUNK, SCHUNK)], db)
        pltpu.sync_copy(db, shared.at[idx2d.at[k]], add=True)
    plsc.subcore_barrier()

    pltpu.sync_copy(shared.at[rows], parts_hbm.at[cid, rows])


def _sc_scatter(oute, dst2d, zeros_hbm):
    mesh = plsc.VectorSubcoreMesh(core_axis_name="c", subcore_axis_name="s")
    f32 = jnp.float32
    out_type = jax.ShapeDtypeStruct((SC_CORES, N_NODES, NS), f32)
    scratch = [
        pltpu.VMEM_SHARED((N_NODES, NS), f32),
        pltpu.VMEM((SCHUNKS_PER_WORKER, SCHUNK), jnp.int32),
        pltpu.VMEM((SCHUNK, NS), f32),
    ]
    fn = pl.kernel(_scatter_body, out_type=out_type, mesh=mesh,
                   scratch_types=scratch,
                   compiler_params=pltpu.CompilerParams(
                       use_tc_tiling_on_sc=False))
    return fn(oute, dst2d, zeros_hbm)


def _combine_body(h_ref, p0_ref, p1_ref, out_ref):
    out_ref[...] = h_ref[...] + (p0_ref[...] + p1_ref[...]) * 0.25


def _combine(h, p0, p1):
    grid = N_NODES // NB
    spec = pl.BlockSpec((NB, NS), lambda i: (i, 0))
    return pl.pallas_call(
        _combine_body,
        grid=(grid,),
        in_specs=[spec, spec, spec],
        out_specs=spec,
        out_shape=jax.ShapeDtypeStruct((N_NODES, NS), jnp.float32),
    )(h, p0, p1)


def kernel(x, pos, edge_attr, node_sigma_emb, edge_index,
           W_ne1, b_ne1, W_ne2, b_ne2, W_ee1, b_ee1, W_ee2, b_ee2,
           W_g1, b_g1, W_g2, b_g2, W_tp):
    f32 = jnp.float32
    src = edge_index[0]
    dst = edge_index[1]

    pos16 = jnp.concatenate(
        [pos.astype(f32), jnp.zeros((N_NODES, 13), f32)], axis=1)
    b_ne1r = b_ne1.reshape(1, NS)
    b_ne2r = b_ne2.reshape(1, NS)
    b_ee1r = b_ee1.reshape(1, NS)
    b_ee2r = b_ee2.reshape(1, NS)
    b_g1r = b_g1.reshape(1, 3 * NS)
    b_g2r = b_g2.reshape(1, 9)

    import numpy as np
    m1 = np.zeros((16, 9), np.float32)
    m2 = np.zeros((16, 9), np.float32)
    alin = np.zeros((16, 9), np.float32)
    kc = np.zeros((1, 9), np.float32)
    sq3, sq15, sq5 = float(_SQ3), float(_SQ15), float(_SQ5)
    kc[0, 0] = 1.0
    alin[0, 1] = sq3
    alin[1, 2] = sq3
    alin[2, 3] = sq3
    m1[0, 4] = sq15; m2[1, 4] = 1.0
    m1[1, 5] = sq15; m2[2, 5] = 1.0
    m1[2, 6] = 1.5 * sq5; m2[2, 6] = 1.0
    kc[0, 6] = -0.5 * sq5
    m1[0, 7] = sq15; m2[2, 7] = 1.0
    m1[0, 8] = 0.5 * sq15; m1[1, 8] = -0.5 * sq15
    m2[0, 8] = 1.0; m2[1, 8] = 1.0
    e48 = np.zeros((NS, NS * 9), np.float32)
    t9 = np.zeros((9, NS * 9), np.float32)
    for i in range(NS):
        for j in range(9):
            e48[i, i * 9 + j] = 1.0
            t9[j, i * 9 + j] = 1.0
    m1, m2, alin, kc, e48, t9 = map(jnp.asarray, (m1, m2, alin, kc, e48, t9))

    h = _node_mlp(x, node_sigma_emb, W_ne1, b_ne1r, W_ne2, b_ne2r)

    hs, hd, sgs, pss, pds = _sc_gather(h, node_sigma_emb, pos16, src, dst)

    oute = _edge_compute(edge_attr, sgs, pss, pds, hs, hd,
                         W_ee1, b_ee1r, W_ee2, b_ee2r,
                         W_g1, b_g1r, W_g2, b_g2r, W_tp,
                         m1, m2, alin, kc, e48, t9)

    dst2d = dst.reshape(N_EDGES // SCHUNK, SCHUNK)
    zeros_hbm = jnp.zeros((N_NODES, NS), f32)
    parts = _sc_scatter(oute, dst2d, zeros_hbm)

    return _combine(h, parts[0], parts[1])


# speedup vs baseline: 3.4721x; 1.2469x over previous
import functools

import numpy as np

import jax
import jax.numpy as jnp
from jax import lax
from jax.experimental import pallas as pl
from jax.experimental.pallas import tpu as pltpu
from jax.experimental.pallas import tpu_sc as plsc

N_NODES = 10000
N_EDGES = 160000
NS = 48

SC_CORES = 2
SC_SUBCORES = 16
SC_WORKERS = SC_CORES * SC_SUBCORES
EDGES_PER_WORKER = N_EDGES // SC_WORKERS
GCHUNK = 200
SCHUNK = 40
SCHUNKS_PER_WORKER = EDGES_PER_WORKER // SCHUNK
NODES_PER_TILE = N_NODES // SC_SUBCORES

EB = 1000
NB = 2000

_SQ3 = 3.0 ** 0.5
_SQ15 = 15.0 ** 0.5
_SQ5 = 5.0 ** 0.5
_SMEAR_STEP = 5.0 / 31.0
_SMEAR_COEFF = -0.5 / (_SMEAR_STEP * _SMEAR_STEP)


def _node_mlp_body(x_ref, sg_ref, p_ref, w1_ref, b1_ref, w2_ref, b2_ref,
                   wsb_ref, tsrc_ref, tdst_ref):
    f32 = jnp.float32
    sg = sg_ref[...]
    cat = jnp.concatenate([x_ref[...], sg], axis=1)
    a = jnp.dot(cat, w1_ref[...], preferred_element_type=f32) + b1_ref[...]
    a = jnp.maximum(a, 0.0)
    h = jnp.dot(a, w2_ref[...], preferred_element_type=f32) + b2_ref[...]
    s = jnp.dot(sg, wsb_ref[...], preferred_element_type=f32)
    p = p_ref[...]
    z16 = jnp.zeros((NB, 16), f32)
    z64 = jnp.zeros((NB, 64), f32)
    tsrc_ref[...] = jnp.concatenate([h, s, p, z16], axis=1)
    tdst_ref[...] = jnp.concatenate([h, p, z64], axis=1)


def _node_mlp(x, sig, pos16, w1, b1, w2, b2, wsb):
    grid = N_NODES // NB
    full = lambda shape: pl.BlockSpec(shape, lambda i: (0, 0))
    return pl.pallas_call(
        _node_mlp_body,
        grid=(grid,),
        in_specs=[
            pl.BlockSpec((NB, 16), lambda i: (i, 0)),
            pl.BlockSpec((NB, 32), lambda i: (i, 0)),
            pl.BlockSpec((NB, 16), lambda i: (i, 0)),
            full((48, NS)),
            full((1, NS)),
            full((NS, NS)),
            full((1, NS)),
            full((32, NS)),
        ],
        out_specs=[
            pl.BlockSpec((NB, 128), lambda i: (i, 0)),
            pl.BlockSpec((NB, 128), lambda i: (i, 0)),
        ],
        out_shape=[
            jax.ShapeDtypeStruct((N_NODES, 128), jnp.float32),
            jax.ShapeDtypeStruct((N_NODES, 128), jnp.float32),
        ],
    )(x, sig, pos16, w1, b1, w2, b2, wsb)


def _gather_body(tsrc_hbm, tdst_hbm, src_hbm, dst_hbm,
                 a_hbm, b_hbm, ichunk, bufa, bufb, sem):
    wid = lax.axis_index("c") * SC_SUBCORES + lax.axis_index("s")
    base = wid * EDGES_PER_WORKER
    for k in range(EDGES_PER_WORKER // GCHUNK):
        sl = pl.ds(base + k * GCHUNK, GCHUNK)
        pltpu.sync_copy(src_hbm.at[sl], ichunk)
        pltpu.async_copy(tsrc_hbm.at[ichunk], bufa, sem).wait()
        pltpu.sync_copy(bufa, a_hbm.at[sl])
        pltpu.sync_copy(dst_hbm.at[sl], ichunk)
        pltpu.async_copy(tdst_hbm.at[ichunk], bufb, sem).wait()
        pltpu.sync_copy(bufb, b_hbm.at[sl])


def _sc_gather(tsrc, tdst, src, dst):
    mesh = plsc.VectorSubcoreMesh(core_axis_name="c", subcore_axis_name="s")
    f32 = jnp.float32
    out_type = [
        jax.ShapeDtypeStruct((N_EDGES, 128), f32),
        jax.ShapeDtypeStruct((N_EDGES, 128), f32),
    ]
    scratch = [
        pltpu.VMEM((GCHUNK,), jnp.int32),
        pltpu.VMEM((GCHUNK, 128), f32),
        pltpu.VMEM((GCHUNK, 128), f32),
        pltpu.SemaphoreType.DMA,
    ]
    fn = pl.kernel(_gather_body, out_type=out_type, mesh=mesh,
                   scratch_types=scratch,
                   compiler_params=pltpu.CompilerParams(
                       use_tc_tiling_on_sc=False))
    return fn(tsrc, tdst, src, dst)


def _edge_body(ea_ref, a_ref, b_ref,
               w1a_ref, w1c_ref, bee1_ref, wee2_ref, bee2_ref,
               wg1_ref, bg1_ref, wg2_ref, bg2_ref, wtp_ref,
               m1_ref, m2_ref, alin_ref, kc_ref, e48_ref, t9_ref, out_ref):
    f32 = jnp.float32
    a = a_ref[...]
    b = b_ref[...]
    hs = a[:, 0:48]
    ssrc = a[:, 48:96]
    hd = b[:, 0:48]
    v = b[:, 48:64] - a[:, 96:112]
    vv = v * v
    ones16 = jnp.full((16, 16), 1.0, f32)
    ones32 = jnp.full((16, 32), 1.0, f32)
    d2_16 = jnp.dot(vv, ones16, preferred_element_type=f32) + 1e-12
    d2_32 = jnp.dot(vv, ones32, preferred_element_type=f32) + 1e-12
    u = v * lax.rsqrt(d2_16)

    offs = lax.broadcasted_iota(jnp.int32, (1, 32), 1).astype(f32) * _SMEAR_STEP
    dd = jnp.sqrt(d2_32) - offs
    demb = jnp.exp(_SMEAR_COEFF * dd * dd)

    e1 = (jnp.dot(ea_ref[...], w1a_ref[...], preferred_element_type=f32)
          + ssrc
          + jnp.dot(demb, w1c_ref[...], preferred_element_type=f32)
          + bee1_ref[...])
    e1 = jnp.maximum(e1, 0.0)
    e2 = jnp.dot(e1, wee2_ref[...], preferred_element_type=f32) + bee2_ref[...]

    g_in = jnp.concatenate([e2, hs, hd], axis=1)
    g1 = jnp.dot(g_in, wg1_ref[...], preferred_element_type=f32) + bg1_ref[...]
    g1 = jnp.maximum(g1, 0.0)
    w9 = jnp.dot(g1, wg2_ref[...], preferred_element_type=f32) + bg2_ref[...]

    sh9 = (jnp.dot(u, m1_ref[...], preferred_element_type=f32)
           * jnp.dot(u, m2_ref[...], preferred_element_type=f32)
           + jnp.dot(u, alin_ref[...], preferred_element_type=f32)
           + kc_ref[...])
    shw = sh9 * w9

    hse = jnp.dot(hs, e48_ref[...], preferred_element_type=f32)
    shwt = jnp.dot(shw, t9_ref[...], preferred_element_type=f32)
    oute = jnp.dot(hse * shwt, wtp_ref[...], preferred_element_type=f32)
    out_ref[...] = jnp.concatenate([oute, jnp.zeros((EB, 80), f32)], axis=1)


def _edge_compute(ea, a, b, w1a, w1c, bee1, wee2, bee2,
                  wg1, bg1, wg2, bg2, wtp, m1, m2, alin, kc, e48, t9):
    grid = N_EDGES // EB
    full = lambda shape: pl.BlockSpec(shape, lambda i: (0, 0))
    eb = lambda d: pl.BlockSpec((EB, d), lambda i: (i, 0))
    return pl.pallas_call(
        _edge_body,
        grid=(grid,),
        in_specs=[
            eb(4), eb(128), eb(128),
            full((4, NS)), full((32, NS)), full((1, NS)),
            full((NS, NS)), full((1, NS)),
            full((3 * NS, 3 * NS)), full((1, 3 * NS)),
            full((3 * NS, 9)), full((1, 9)),
            full((NS * 9, NS)),
            full((16, 9)), full((16, 9)), full((16, 9)), full((1, 9)),
            full((NS, NS * 9)), full((9, NS * 9)),
        ],
        out_specs=pl.BlockSpec((EB, 128), lambda i: (i, 0)),
        out_shape=jax.ShapeDtypeStruct((N_EDGES, 128), jnp.float32),
    )(ea, a, b, w1a, w1c, bee1, wee2, bee2, wg1, bg1, wg2, bg2, wtp,
      m1, m2, alin, kc, e48, t9)


def _scatter_body(oute_hbm, dst2d_hbm, zero_hbm, parts_hbm,
                  shared, idx2d, db):
    cid = lax.axis_index("c")
    sid = lax.axis_index("s")
    wid = cid * SC_SUBCORES + sid
    rows = pl.ds(sid * NODES_PER_TILE, NODES_PER_TILE)

    pltpu.sync_copy(zero_hbm.at[rows], shared.at[rows])
    pltpu.sync_copy(
        dst2d_hbm.at[pl.ds(wid * SCHUNKS_PER_WORKER, SCHUNKS_PER_WORKER)],
        idx2d)
    plsc.subcore_barrier()

    base = wid * EDGES_PER_WORKER
    for k in range(SCHUNKS_PER_WORKER):
        pltpu.sync_copy(oute_hbm.at[pl.ds(base + k * SCHUNK, SCHUNK)], db)
        pltpu.sync_copy(db, shared.at[idx2d.at[k]], add=True)
    plsc.subcore_barrier()

    pltpu.sync_copy(shared.at[rows], parts_hbm.at[cid, rows])


def _sc_scatter(oute, dst2d, zeros_hbm):
    mesh = plsc.VectorSubcoreMesh(core_axis_name="c", subcore_axis_name="s")
    f32 = jnp.float32
    out_type = jax.ShapeDtypeStruct((SC_CORES, N_NODES, 128), f32)
    scratch = [
        pltpu.VMEM_SHARED((N_NODES, 128), f32),
        pltpu.VMEM((SCHUNKS_PER_WORKER, SCHUNK), jnp.int32),
        pltpu.VMEM((SCHUNK, 128), f32),
    ]
    fn = pl.kernel(_scatter_body, out_type=out_type, mesh=mesh,
                   scratch_types=scratch,
                   compiler_params=pltpu.CompilerParams(
                       use_tc_tiling_on_sc=False))
    return fn(oute, dst2d, zeros_hbm)


def _combine_body(ts_ref, p0_ref, p1_ref, out_ref):
    h = ts_ref[:, 0:48]
    p0 = p0_ref[0, :, 0:48]
    p1 = p1_ref[0, :, 0:48]
    out_ref[...] = h + (p0 + p1) * 0.25


def _combine(tsrc, parts):
    grid = N_NODES // NB
    return pl.pallas_call(
        _combine_body,
        grid=(grid,),
        in_specs=[
            pl.BlockSpec((NB, 128), lambda i: (i, 0)),
            pl.BlockSpec((1, NB, 128), lambda i: (0, i, 0)),
            pl.BlockSpec((1, NB, 128), lambda i: (1, i, 0)),
        ],
        out_specs=pl.BlockSpec((NB, NS), lambda i: (i, 0)),
        out_shape=jax.ShapeDtypeStruct((N_NODES, NS), jnp.float32),
    )(tsrc, parts, parts)


def kernel(x, pos, edge_attr, node_sigma_emb, edge_index,
           W_ne1, b_ne1, W_ne2, b_ne2, W_ee1, b_ee1, W_ee2, b_ee2,
           W_g1, b_g1, W_g2, b_g2, W_tp):
    f32 = jnp.float32
    src = edge_index[0]
    dst = edge_index[1]

    pos16 = jnp.concatenate(
        [pos.astype(f32), jnp.zeros((N_NODES, 13), f32)], axis=1)
    b_ne1r = b_ne1.reshape(1, NS)
    b_ne2r = b_ne2.reshape(1, NS)
    b_ee1r = b_ee1.reshape(1, NS)
    b_ee2r = b_ee2.reshape(1, NS)
    b_g1r = b_g1.reshape(1, 3 * NS)
    b_g2r = b_g2.reshape(1, 9)
    w1a = W_ee1[0:4]
    wsb = W_ee1[4:36]
    w1c = W_ee1[36:68]

    m1 = np.zeros((16, 9), np.float32)
    m2 = np.zeros((16, 9), np.float32)
    alin = np.zeros((16, 9), np.float32)
    kc = np.zeros((1, 9), np.float32)
    sq3, sq15, sq5 = float(_SQ3), float(_SQ15), float(_SQ5)
    kc[0, 0] = 1.0
    alin[0, 1] = sq3
    alin[1, 2] = sq3
    alin[2, 3] = sq3
    m1[0, 4] = sq15; m2[1, 4] = 1.0
    m1[1, 5] = sq15; m2[2, 5] = 1.0
    m1[2, 6] = 1.5 * sq5; m2[2, 6] = 1.0
    kc[0, 6] = -0.5 * sq5
    m1[0, 7] = sq15; m2[2, 7] = 1.0
    m1[0, 8] = 0.5 * sq15; m1[1, 8] = -0.5 * sq15
    m2[0, 8] = 1.0; m2[1, 8] = 1.0
    e48 = np.zeros((NS, NS * 9), np.float32)
    t9 = np.zeros((9, NS * 9), np.float32)
    for i in range(NS):
        for j in range(9):
            e48[i, i * 9 + j] = 1.0
            t9[j, i * 9 + j] = 1.0
    m1, m2, alin, kc, e48, t9 = map(jnp.asarray, (m1, m2, alin, kc, e48, t9))

    tsrc, tdst = _node_mlp(x, node_sigma_emb, pos16,
                           W_ne1, b_ne1r, W_ne2, b_ne2r, wsb)

    a, b = _sc_gather(tsrc, tdst, src, dst)

    oute = _edge_compute(edge_attr, a, b,
                         w1a, w1c, b_ee1r, W_ee2, b_ee2r,
                         W_g1, b_g1r, W_g2, b_g2r, W_tp,
                         m1, m2, alin, kc, e48, t9)

    dst2d = dst.reshape(N_EDGES // SCHUNK, SCHUNK)
    zeros_hbm = jnp.zeros((N_NODES, 128), f32)
    parts = _sc_scatter(oute, dst2d, zeros_hbm)

    return _combine(tsrc, parts)
